# Initial kernel scaffold; baseline (speedup 1.0000x reference)
import jax
import jax.numpy as jnp
from jax.experimental import pallas as pl


def kernel(x1, x2, out_g, in_g, out_weight, in_weight, bias1, bias2):
    raise NotImplementedError("write your pallas kernel here")


# speedup vs baseline: 1.0079x; 1.0079x over previous
import jax
import jax.numpy as jnp
from jax.experimental import pallas as pl
from jax.experimental.pallas import tpu as pltpu

_TM = 512


def _fused_kernel(x_ref, w_ref, g_ref, b_ref, o_ref, xw_ref):
    i = pl.program_id(1)

    @pl.when(i == 0)
    def _proj():
        xw_ref[...] = jnp.dot(
            x_ref[0], w_ref[0], preferred_element_type=jnp.float32
        ).astype(jnp.bfloat16)

    acc = jnp.dot(g_ref[0], xw_ref[...], preferred_element_type=jnp.float32)
    o_ref[0] = jnp.maximum(acc + b_ref[0], 0.0)


def kernel(x1, x2, out_g, in_g, out_weight, in_weight, bias1, bias2):
    n, d = x1.shape
    tm = _TM if n % _TM == 0 else n

    x = jnp.stack([x1, x2]).astype(jnp.bfloat16)
    w = jnp.stack([out_weight, in_weight]).astype(jnp.bfloat16)
    g = jnp.stack([out_g, in_g]).astype(jnp.bfloat16)
    b = jnp.stack([bias1, bias2]).reshape(2, 1, d)

    out = pl.pallas_call(
        _fused_kernel,
        out_shape=jax.ShapeDtypeStruct((2, n, d), jnp.float32),
        grid_spec=pltpu.PrefetchScalarGridSpec(
            num_scalar_prefetch=0,
            grid=(2, n // tm),
            in_specs=[
                pl.BlockSpec((1, n, d), lambda bb, i: (bb, 0, 0)),
                pl.BlockSpec((1, d, d), lambda bb, i: (bb, 0, 0)),
                pl.BlockSpec((1, tm, n), lambda bb, i: (bb, i, 0)),
                pl.BlockSpec((1, 1, d), lambda bb, i: (bb, 0, 0)),
            ],
            out_specs=pl.BlockSpec((1, tm, d), lambda bb, i: (bb, i, 0)),
            scratch_shapes=[pltpu.VMEM((n, d), jnp.bfloat16)],
        ),
        compiler_params=pltpu.CompilerParams(
            dimension_semantics=("parallel", "arbitrary"),
        ),
        cost_estimate=pl.CostEstimate(
            flops=2 * (2 * n * d * d + 2 * n * n * d),
            transcendentals=0,
            bytes_accessed=2 * (2 * n * d + 2 * d * d + 2 * n * n) + 4 * 2 * n * d,
        ),
    )(x, w, g, b)
    return out[0], out[1]


# speedup vs baseline: 1.8791x; 1.8645x over previous
import jax
import jax.numpy as jnp
from jax.experimental import pallas as pl
from jax.experimental.pallas import tpu as pltpu

_TM = 512


def _fused_kernel(x1_ref, x2_ref, w1_ref, w2_ref, g1_ref, g2_ref,
                  b1_ref, b2_ref, o1_ref, o2_ref, xw1_ref, xw2_ref):
    i = pl.program_id(1)
    bb = pl.program_id(2)

    @pl.when((i == 0) & (bb == 0))
    def _proj1():
        xw1_ref[...] = jnp.dot(
            x1_ref[...].astype(jnp.bfloat16),
            w1_ref[...].astype(jnp.bfloat16),
            preferred_element_type=jnp.float32,
        ).astype(jnp.bfloat16)

    @pl.when((i == 0) & (bb == 1))
    def _proj2():
        xw2_ref[...] = jnp.dot(
            x2_ref[...].astype(jnp.bfloat16),
            w2_ref[...].astype(jnp.bfloat16),
            preferred_element_type=jnp.float32,
        ).astype(jnp.bfloat16)

    @pl.when(bb == 0)
    def _prop1():
        acc = jnp.dot(g1_ref[...].astype(jnp.bfloat16), xw1_ref[...],
                      preferred_element_type=jnp.float32)
        o1_ref[...] = jnp.maximum(acc + b1_ref[...], 0.0)

    @pl.when(bb == 1)
    def _prop2():
        acc = jnp.dot(g2_ref[...].astype(jnp.bfloat16), xw2_ref[...],
                      preferred_element_type=jnp.float32)
        o2_ref[...] = jnp.maximum(acc + b2_ref[...], 0.0)


def kernel(x1, x2, out_g, in_g, out_weight, in_weight, bias1, bias2):
    n, d = x1.shape
    tm = _TM if n % (2 * _TM) == 0 else n
    nb2 = n // (2 * tm) if n % (2 * tm) == 0 else 1

    b1 = bias1.reshape(1, d)
    b2 = bias2.reshape(1, d)

    def row(h, i, bb):
        del bb
        return (h * nb2 + i, 0)

    const = lambda h, i, bb: (0, 0)

    out1, out2 = pl.pallas_call(
        _fused_kernel,
        out_shape=(
            jax.ShapeDtypeStruct((n, d), jnp.float32),
            jax.ShapeDtypeStruct((n, d), jnp.float32),
        ),
        grid_spec=pltpu.PrefetchScalarGridSpec(
            num_scalar_prefetch=0,
            grid=(2, nb2, 2),
            in_specs=[
                pl.BlockSpec((n, d), const),
                pl.BlockSpec((n, d), const),
                pl.BlockSpec((d, d), const),
                pl.BlockSpec((d, d), const),
                pl.BlockSpec((tm, n), row),
                pl.BlockSpec((tm, n), row),
                pl.BlockSpec((1, d), const),
                pl.BlockSpec((1, d), const),
            ],
            out_specs=(
                pl.BlockSpec((tm, d), row),
                pl.BlockSpec((tm, d), row),
            ),
            scratch_shapes=[
                pltpu.VMEM((n, d), jnp.bfloat16),
                pltpu.VMEM((n, d), jnp.bfloat16),
            ],
        ),
        compiler_params=pltpu.CompilerParams(
            dimension_semantics=("parallel", "arbitrary", "arbitrary"),
        ),
        cost_estimate=pl.CostEstimate(
            flops=2 * (2 * n * d * d + 2 * n * n * d),
            transcendentals=0,
            bytes_accessed=4 * (2 * n * n + 2 * n * d + 2 * d * d + 2 * n * d),
        ),
    )(x1, x2, out_weight, in_weight, out_g, in_g, b1, b2)
    return out1, out2
